# baseline (speedup 1.0000x reference)
import jax
import jax.numpy as jnp
from jax.experimental import pallas as pl
from jax.experimental.pallas import tpu as pltpu

_H = 384
_W = 384
_N = float(_H * _W)
_R = _H // 8
_CPB = 8
_NB = 96 // _CPB


def _top10_sum_fallback(xs_ref, c, tau):
    def step(_, carry):
        bound, acc, rem = carry
        v = xs_ref[c]
        w = jnp.where((v >= tau) & (v < bound), v, -jnp.inf)
        mx = jnp.max(w)
        cc = jnp.sum(jnp.where(w == mx, 1.0, 0.0))
        take = jnp.minimum(cc, rem)
        acc = acc + jnp.where(take > 0.0, take * mx, 0.0)
        rem = rem - take
        return mx, acc, rem

    _, acc, _ = jax.lax.fori_loop(
        0, 10, step,
        (jnp.float32(jnp.inf), jnp.float32(0.0), jnp.float32(10.0))
    )
    return acc


def _fold3(a):
    return jnp.maximum(jnp.maximum(a[:, :128], a[:, 128:256]), a[:, 256:])


def _fold3s(a):
    return a[:, :128] + a[:, 128:256] + a[:, 256:]


def _step(i, x_ref, m2_ref, m2g_ref, cef_ref, out_ref,
          xsA_ref, wsvA_ref, accA_ref, xsB_ref, wsvB_ref, accB_ref):
    def tau_step(_, carry):
        out = []
        for c in range(_CPB):
            W, _tau = carry[c]
            mx = jnp.max(W, axis=1, keepdims=True)
            mx = jnp.max(mx, axis=0, keepdims=True)
            mxb = jax.lax.broadcast_in_dim(mx, (8, 128), (0, 1))
            W = jnp.where(W == mxb, -jnp.inf, W)
            out.append((W, mxb))
        return tuple(out)

    Ws = [wsvB_ref[c] for c in range(_CPB)]
    taus_c = jax.lax.fori_loop(
        0, 10, tau_step,
        tuple((Ws[c], Ws[c]) for c in range(_CPB)),
        unroll=True,
    )
    tau_wide = [
        jnp.concatenate([taus_c[c][1]] * (_W // 128), axis=1)
        for c in range(_CPB)
    ]

    sums_prev = [
        (jnp.sum(accB_ref[c, 0]), jnp.sum(accB_ref[c, 1]),
         jnp.sum(accB_ref[c, 2]), jnp.sum(accB_ref[c, 3]))
        for c in range(_CPB)
    ]

    def chunk(j, carry):
        main_c, filt_c = carry
        m2c = m2_ref[0, j]
        m2gc = m2g_ref[0, j]
        mains = []
        filts = []
        for c in range(_CPB):
            aS, aA, aB, aD, aM = main_c[c]
            xv = x_ref[c, j]
            xsA_ref[c, j] = xv
            vm2 = xv * m2c
            aS = aS + _fold3s(xv)
            aA = aA + _fold3s(xv * vm2)
            aB = aB + _fold3s(vm2)
            aD = aD + _fold3s(xv * m2gc)
            aM = jnp.maximum(aM, xv)
            mains.append((aS, aA, aB, aD, aM))

            aC, aV = filt_c[c]
            pv = xsB_ref[c, j]
            sel = pv >= tau_wide[c]
            aC = aC + _fold3s(jnp.where(sel, 1.0, 0.0))
            aV = aV + _fold3s(jnp.where(sel, pv, 0.0))
            filts.append((aC, aV))
        return tuple(mains), tuple(filts)

    zero1 = jnp.zeros((8, 128), jnp.float32)
    main_init = tuple(
        (zero1, zero1, zero1, zero1,
         jnp.full((8, _W), -jnp.inf, jnp.float32))
        for _ in range(_CPB)
    )
    filt_init = tuple((zero1, zero1) for _ in range(_CPB))
    mains, filts = jax.lax.fori_loop(
        0, _R, chunk, (main_init, filt_init), unroll=True)

    for c in range(_CPB):
        aS, aA, aB, aD, M = mains[c]
        wsvA_ref[c] = _fold3(M)
        accA_ref[c, 0] = aS
        accA_ref[c, 1] = aA
        accA_ref[c, 2] = aB
        accA_ref[c, 3] = aD

    C = cef_ref[0]
    E = cef_ref[1]
    F = cef_ref[2]

    cnts = [jnp.sum(filts[c][0]) for c in range(_CPB)]
    ssums = [jnp.sum(filts[c][1]) for c in range(_CPB)]

    all_exact = (cnts[0] == 10.0)
    for c in range(1, _CPB):
        all_exact = all_exact & (cnts[c] == 10.0)
    all_exact = all_exact | (i == 0)

    def _common(_):
        return tuple(ssums)

    def _rare(_):
        out = []
        for c in range(_CPB):
            tau_s = taus_c[c][1][0, 0]
            out.append(jax.lax.cond(
                cnts[c] == 10.0, lambda _, cc=c: ssums[cc],
                lambda _, cc=c, ts=tau_s: _top10_sum_fallback(
                    xsB_ref, cc, ts),
                operand=None))
        return tuple(out)

    top10_sums = jax.lax.cond(all_exact, _common, _rare, operand=None)

    loss = jnp.float32(0.0)
    for c in range(_CPB):
        top10_sum = top10_sums[c]
        S, A, B, D = sums_prev[c]
        mu = S / _N
        max_avg = top10_sum / 10.0 - mu
        denom = jnp.where(max_avg < 1e-20, max_avg + 1e-19, max_avg)
        num = ((A - 2.0 * mu * B + mu * mu * C) / denom
               - 2.0 * (D - mu * E)) / denom + F
        loss = loss + num / _N

    loss = jnp.where(i > 0, loss, 0.0)
    out_ref[...] += jnp.full(out_ref.shape, loss, dtype=jnp.float32)


def _body(x_ref, gt_ref, m_ref, out_ref,
          m2_ref, m2g_ref, cef_ref,
          xs0_ref, xs1_ref, wsv0_ref, wsv1_ref, acc0_ref, acc1_ref):
    i = pl.program_id(0)
    par = jax.lax.rem(i, 2)

    @pl.when(i == 0)
    def _():
        m = m_ref[0]
        g = gt_ref[0]
        m2 = m * m
        m2g = m2 * g
        m2_ref[0] = m2
        m2g_ref[0] = m2g
        cef_ref[0] = jnp.sum(m2)
        cef_ref[1] = jnp.sum(m2g)
        cef_ref[2] = jnp.sum(m2g * g)
        out_ref[...] = jnp.zeros(out_ref.shape, jnp.float32)

    @pl.when(par == 0)
    def _():
        _step(i, x_ref, m2_ref, m2g_ref, cef_ref, out_ref,
              xs0_ref, wsv0_ref, acc0_ref, xs1_ref, wsv1_ref, acc1_ref)

    @pl.when(par == 1)
    def _():
        _step(i, x_ref, m2_ref, m2g_ref, cef_ref, out_ref,
              xs1_ref, wsv1_ref, acc1_ref, xs0_ref, wsv0_ref, acc0_ref)


@jax.jit
def kernel(pattern, pattern_gt, mask):
    ch = pattern.shape[1]
    x = pattern.reshape(ch, _R, 8, _W)
    out = pl.pallas_call(
        _body,
        grid=(_NB + 1,),
        in_specs=[
            pl.BlockSpec((_CPB, _R, 8, _W),
                         lambda i: (jnp.minimum(i, _NB - 1), 0, 0, 0)),
            pl.BlockSpec((1, _R, 8, _W), lambda i: (0, 0, 0, 0)),
            pl.BlockSpec((1, _R, 8, _W), lambda i: (0, 0, 0, 0)),
        ],
        out_specs=pl.BlockSpec((8, 128), lambda i: (0, 0)),
        out_shape=jax.ShapeDtypeStruct((8, 128), jnp.float32),
        scratch_shapes=[
            pltpu.VMEM((1, _R, 8, _W), jnp.float32),
            pltpu.VMEM((1, _R, 8, _W), jnp.float32),
            pltpu.SMEM((3,), jnp.float32),
            pltpu.VMEM((_CPB, _R, 8, _W), jnp.float32),
            pltpu.VMEM((_CPB, _R, 8, _W), jnp.float32),
            pltpu.VMEM((_CPB, 8, 128), jnp.float32),
            pltpu.VMEM((_CPB, 8, 128), jnp.float32),
            pltpu.VMEM((_CPB, 4, 8, 128), jnp.float32),
            pltpu.VMEM((_CPB, 4, 8, 128), jnp.float32),
        ],
    )(x, pattern_gt.reshape(1, _R, 8, _W), mask.reshape(1, _R, 8, _W))
    return out[0, 0].reshape(1)
